# baseline (speedup 1.0000x reference)
import functools

import jax
import jax.numpy as jnp
from jax import lax
from jax.experimental import pallas as pl

VOCAB = 1000
EMBED_DIM = 64
HIDDEN_DIM = 32


LBLK = 2


def _mlp_t_body(xt_ref, embt_ref, w1t_ref, b1_ref, w2t_ref, b2_ref, out_ref):
    for i in range(LBLK):
        xv = xt_ref[i]
        iota_v = lax.broadcasted_iota(jnp.int32, (VOCAB, xv.shape[1]), 0)
        m = jnp.where(iota_v == xv, 1.0, 0.0)
        et = lax.dot_general(
            embt_ref[...], m, (((1,), (0,)), ((), ())),
            preferred_element_type=jnp.float32)
        ht = lax.dot_general(
            w1t_ref[...], et, (((1,), (0,)), ((), ())),
            preferred_element_type=jnp.float32)
        ht = jnp.maximum(ht + b1_ref[...], 0.0)
        ot = lax.dot_general(
            w2t_ref[...], ht, (((1,), (0,)), ((), ())),
            preferred_element_type=jnp.float32) + b2_ref[...]
        out_ref[i] = ot


@functools.cache
def _make_mlp_t(B, L, V):
    return pl.pallas_call(
        _mlp_t_body,
        grid=(L // LBLK,),
        in_specs=[
            pl.BlockSpec((LBLK, 1, B), lambda l: (l, 0, 0)),
            pl.BlockSpec((EMBED_DIM, V), lambda l: (0, 0)),
            pl.BlockSpec((HIDDEN_DIM, EMBED_DIM), lambda l: (0, 0)),
            pl.BlockSpec((HIDDEN_DIM, 1), lambda l: (0, 0)),
            pl.BlockSpec((V, HIDDEN_DIM), lambda l: (0, 0)),
            pl.BlockSpec((V, 1), lambda l: (0, 0)),
        ],
        out_specs=pl.BlockSpec((LBLK, V, B), lambda l: (l, 0, 0)),
        out_shape=jax.ShapeDtypeStruct((L, V, B), jnp.float32),
    )


def kernel(x, emb, W1, b1, W2, b2):
    Bt, L = x.shape
    xt = x.astype(jnp.int32).T.reshape(L, 1, Bt)
    ot = _make_mlp_t(Bt, L, VOCAB)(
        xt, emb.T, W1.T, b1.reshape(HIDDEN_DIM, 1),
        W2.T, b2.reshape(VOCAB, 1))
    return jnp.transpose(ot, (2, 0, 1))
